# Initial kernel scaffold; baseline (speedup 1.0000x reference)
import jax
import jax.numpy as jnp
from jax.experimental import pallas as pl


def kernel(edge_index, edge_type, positive_triples, corrupted_tails, params):
    raise NotImplementedError("write your pallas kernel here")


# speedup vs baseline: 1.6260x; 1.6260x over previous
import functools

import jax
import jax.numpy as jnp
from jax import lax
from jax.experimental import pallas as pl
from jax.experimental.pallas import tpu as pltpu
from jax.experimental.pallas import tpu_sc as plsc

N_NODES = 10000
N_REL = 8
DIM = 256
N_LAYERS = 3
N_POS = 1024

NC, NS = 2, 16
NW = NC * NS
CHUNK = 64
SCAN = 2048
EDGE_PAD = 80 * SCAN
NPASS = 2
BUCKET = 160
ACCROWS = 168
PDUMP = 160
NBKT = NW * NPASS
CAP = EDGE_PAD + 512
BLK = 512

_PC = pl.pallas_call


def _sc_edge_scan(g_pad, dst_pad):
    mesh = plsc.VectorSubcoreMesh(core_axis_name="c", subcore_axis_name="s")
    niter = EDGE_PAD // SCAN
    PEND = SCAN + 512

    @functools.partial(
        pl.kernel,
        out_type=[jax.ShapeDtypeStruct((NBKT * CAP,), jnp.int32),
                  jax.ShapeDtypeStruct((NBKT, 16), jnp.int32)],
        mesh=mesh,
        compiler_params=pltpu.CompilerParams(needs_layout_passes=False),
        scratch_types=[
            pltpu.VMEM((SCAN,), jnp.int32),
            pltpu.VMEM((SCAN,), jnp.int32),
            pltpu.VMEM((PEND,), jnp.int32),
            pltpu.VMEM((PEND,), jnp.int32),
            pltpu.VMEM((1, 16), jnp.int32),
        ],
    )
    def k(g_hbm, d_hbm, ent_hbm, cnt_hbm, gs_v, ds_v, pa, pb, cv):
        c = lax.axis_index("c")
        s = lax.axis_index("s")
        wid = s * NC + c
        iota16 = lax.iota(jnp.int32, 16)
        lo = wid * NPASS * BUCKET
        bktA = wid * NPASS
        bktB = bktA + 1

        def scan_iter(cc, carry):
            pcA, pcB, flA, flB = carry
            base = cc * SCAN
            pltpu.sync_copy(g_hbm.at[pl.ds(base, SCAN)], gs_v)
            pltpu.sync_copy(d_hbm.at[pl.ds(base, SCAN)], ds_v)

            def vec16(i, pc2):
                pcA2, pcB2 = pc2
                dvec = ds_v[pl.ds(i * 16, 16)]
                gvec = gs_v[pl.ds(i * 16, 16)]
                ldd = dvec - lo
                mA = (ldd >= 0) & (ldd < BUCKET)
                mB = (ldd >= BUCKET) & (ldd < 2 * BUCKET)
                pkA = gvec * 512 + jnp.where(mA, ldd, PDUMP)
                pkB = gvec * 512 + jnp.where(mB, ldd - BUCKET, PDUMP)
                plsc.store_compressed(pa.at[pl.ds(pcA2, 16)], pkA, mask=mA)
                plsc.store_compressed(pb.at[pl.ds(pcB2, 16)], pkB, mask=mB)
                cA = plsc.all_reduce_population_count(mA)
                cB = plsc.all_reduce_population_count(mB)
                return (pcA2 + cA[0], pcB2 + cB[0])

            pcA, pcB = lax.fori_loop(0, SCAN // 16, vec16, (pcA, pcB))

            def flush(pend, bkt, pc, fl):
                nb = pc // 512

                def fbody(j, carry):
                    off = pl.multiple_of(bkt * CAP + fl + j * 512, 512)
                    pltpu.sync_copy(pend.at[pl.ds(j * 512, 512)],
                                    ent_hbm.at[pl.ds(off, 512)])
                    return carry
                lax.fori_loop(0, nb, fbody, 0)
                rb = nb * 512
                for g16 in range(512 // 16):
                    pend[pl.ds(g16 * 16, 16)] = pend[pl.ds(rb + g16 * 16, 16)]
                return pc - rb, fl + rb

            pcA, flA = flush(pa, bktA, pcA, flA)
            pcB, flB = flush(pb, bktB, pcB, flB)
            return (pcA, pcB, flA, flB)

        pcA, pcB, flA, flB = lax.fori_loop(0, niter, scan_iter, (0, 0, 0, 0))

        def finish(pend, bkt, pc, fl):
            for g16 in range(512 // 16):
                lane = g16 * 16 + iota16
                pv = pend[pl.ds(g16 * 16, 16)]
                dumpg = wid * 2048 + lane * 4
                pend[pl.ds(g16 * 16, 16)] = jnp.where(
                    lane < pc, pv, dumpg * 512 + PDUMP)
            off = pl.multiple_of(bkt * CAP + fl, 512)
            pltpu.sync_copy(pend.at[pl.ds(0, 512)],
                            ent_hbm.at[pl.ds(off, 512)])
            cv[0, pl.ds(0, 16)] = jnp.zeros((16,), jnp.int32) + (fl + 512)
            pltpu.sync_copy(cv, cnt_hbm.at[pl.ds(bkt, 1)])

        finish(pa, bktA, pcA, flA)
        finish(pb, bktB, pcB, flB)

    return k(g_pad, dst_pad)


def _sc_edge_acc(tr_flat, entries, counts, want_deg):
    mesh = plsc.VectorSubcoreMesh(core_axis_name="c", subcore_axis_name="s")
    out_type = [jax.ShapeDtypeStruct((NBKT * BUCKET, DIM), jnp.float32)]
    if want_deg:
        out_type.append(jax.ShapeDtypeStruct((NBKT * BUCKET, 16), jnp.float32))

    @functools.partial(
        pl.kernel,
        out_type=out_type,
        mesh=mesh,
        compiler_params=pltpu.CompilerParams(needs_layout_passes=False),
        scratch_types=[
            pltpu.VMEM((512,), jnp.int32),
            pltpu.VMEM((CHUNK,), jnp.int32),
            pltpu.VMEM((CHUNK,), jnp.int32),
            pltpu.VMEM((CHUNK,), jnp.int32),
            pltpu.VMEM((CHUNK,), jnp.int32),
            pltpu.VMEM((CHUNK, DIM), jnp.float32),
            pltpu.VMEM((CHUNK, DIM), jnp.float32),
            pltpu.VMEM((ACCROWS, DIM), jnp.float32),
            pltpu.VMEM((ACCROWS, 16), jnp.float32),
            pltpu.VMEM((1, 16), jnp.int32),
            pltpu.SemaphoreType.DMA,
            pltpu.SemaphoreType.DMA,
        ],
    )
    def k(tr_hbm, ent_hbm, cnt_hbm, *rest):
        if want_deg:
            out_hbm, deg_hbm = rest[:2]
            rest = rest[2:]
        else:
            out_hbm, = rest[:1]
            rest = rest[1:]
        eblk, gi0, gi1, ld0, ld1, r0, r1, acc, dacc, cv, sem0, sem1 = rest
        gi = (gi0, gi1)
        ld = (ld0, ld1)
        rows = (r0, r1)
        sem = (sem0, sem1)
        c = lax.axis_index("c")
        s = lax.axis_index("s")
        wid = s * NC + c
        ones16 = jnp.ones((16,), jnp.float32)
        zeros16 = jnp.zeros((16,), jnp.float32)

        def issue(cidx, slot):
            off = lax.rem(cidx, 8) * CHUNK
            for jj in range(CHUNK // 16):
                pv = eblk[pl.ds(off + jj * 16, 16)]
                gi[slot][pl.ds(jj * 16, 16)] = lax.shift_right_logical(pv, 9)
                ld[slot][pl.ds(jj * 16, 16)] = lax.bitwise_and(pv, 511)
            pltpu.async_copy(tr_hbm.at[gi[slot]], rows[slot], sem[slot])

        def drain_acc(slot):
            pltpu.make_async_copy(tr_hbm.at[gi[slot]], rows[slot],
                                  sem[slot]).wait()

            def grp(g16, carry):
                ldvec = ld[slot][pl.ds(g16 * 16, 16)]
                for j2 in range(16):
                    lde = ldvec[j2]
                    for jj in range(DIM // 16):
                        xv = rows[slot][g16 * 16 + j2, pl.ds(jj * 16, 16)]
                        plsc.addupdate(acc.at[lde, pl.ds(jj * 16, 16)], xv)
                    if want_deg:
                        plsc.addupdate(dacc.at[lde, pl.ds(0, 16)], ones16)
                return carry
            lax.fori_loop(0, CHUNK // 16, grp, 0)

        def pass_body(p, carry):
            bkt = wid * NPASS + p
            lo = bkt * BUCKET

            def zrow(r, carry2):
                for j in range(DIM // 16):
                    acc[r, pl.ds(j * 16, 16)] = zeros16
                dacc[r, pl.ds(0, 16)] = zeros16
                return carry2
            lax.fori_loop(0, ACCROWS, zrow, 0)

            pltpu.sync_copy(cnt_hbm.at[pl.ds(bkt, 1)], cv)
            cvec = cv[0, pl.ds(0, 16)]
            nch = cvec[0] // CHUNK

            pltpu.sync_copy(
                ent_hbm.at[pl.ds(pl.multiple_of(bkt * CAP, 512), 512)], eblk)
            issue(0, 0)

            def pair(q, carry2):
                issue(2 * q + 1, 1)
                drain_acc(0)
                c2 = 2 * q + 2

                @pl.when(c2 < nch)
                def _():
                    @pl.when(lax.rem(c2, 8) == 0)
                    def _():
                        boff = pl.multiple_of(
                            bkt * CAP + (c2 // 8) * 512, 512)
                        pltpu.sync_copy(ent_hbm.at[pl.ds(boff, 512)], eblk)
                    issue(c2, 0)
                drain_acc(1)
                return carry2
            lax.fori_loop(0, nch // 2, pair, 0)

            pltpu.sync_copy(acc.at[pl.ds(0, BUCKET)],
                            out_hbm.at[pl.ds(lo, BUCKET)])
            if want_deg:
                pltpu.sync_copy(dacc.at[pl.ds(0, BUCKET)],
                                deg_hbm.at[pl.ds(lo, BUCKET)])
            return carry

        lax.fori_loop(0, NPASS, pass_body, 0)

    outs = k(tr_flat, entries, counts)
    agg = outs[0][:N_NODES]
    if want_deg:
        return agg, outs[1][:N_NODES, 0]
    return agg


def _sc_gather_rows(table, idx):
    b = idx.shape[0]
    bpw = b // (NC * NS)
    mesh = plsc.VectorSubcoreMesh(core_axis_name="c", subcore_axis_name="s")

    @functools.partial(
        pl.kernel,
        out_type=jax.ShapeDtypeStruct((b, DIM), jnp.float32),
        mesh=mesh,
        compiler_params=pltpu.CompilerParams(needs_layout_passes=False),
        scratch_types=[
            pltpu.VMEM((bpw,), jnp.int32),
            pltpu.VMEM((bpw, DIM), jnp.float32),
            pltpu.SemaphoreType.DMA,
        ],
    )
    def k(tab_hbm, idx_hbm, out_hbm, idx_v, rows_v, sem):
        wid = lax.axis_index("s") * NC + lax.axis_index("c")
        base = wid * bpw
        pltpu.sync_copy(idx_hbm.at[pl.ds(base, bpw)], idx_v)
        pltpu.async_copy(tab_hbm.at[idx_v], rows_v, sem).wait()
        pltpu.sync_copy(rows_v, out_hbm.at[pl.ds(base, bpw)])

    return k(table, idx)


def _tc_rel_transform(h, w):
    br = 2000
    nb = N_NODES // br

    def body(h_ref, w_ref, o_ref):
        o_ref[...] = jnp.dot(h_ref[...], w_ref[0],
                             preferred_element_type=jnp.float32)

    return _PC(
        body,
        grid=(N_REL, nb),
        in_specs=[
            pl.BlockSpec((br, DIM), lambda r, i: (i, 0)),
            pl.BlockSpec((1, DIM, DIM), lambda r, i: (r, 0, 0)),
        ],
        out_specs=pl.BlockSpec((br, DIM), lambda r, i: (r * nb + i, 0)),
        out_shape=jax.ShapeDtypeStruct((N_REL * N_NODES, DIM), jnp.float32),
    )(h, w)


def _tc_layer_update(agg, degc, h, sw, gamma, beta, final_ln):
    br = 2000

    def body(a_ref, d_ref, h_ref, w_ref, g_ref, b_ref, o_ref):
        hn = a_ref[...] / d_ref[...] + jnp.dot(
            h_ref[...], w_ref[...], preferred_element_type=jnp.float32)
        hn = jnp.maximum(hn, 0.0)
        if final_ln:
            mu = jnp.mean(hn, axis=-1, keepdims=True)
            var = jnp.mean((hn - mu) ** 2, axis=-1, keepdims=True)
            hn = (hn - mu) / jnp.sqrt(var + 1e-5) * g_ref[...] + b_ref[...]
        o_ref[...] = hn

    return _PC(
        body,
        grid=(N_NODES // br,),
        in_specs=[
            pl.BlockSpec((br, DIM), lambda i: (i, 0)),
            pl.BlockSpec((br, 1), lambda i: (i, 0)),
            pl.BlockSpec((br, DIM), lambda i: (i, 0)),
            pl.BlockSpec((DIM, DIM), lambda i: (0, 0)),
            pl.BlockSpec((1, DIM), lambda i: (0, 0)),
            pl.BlockSpec((1, DIM), lambda i: (0, 0)),
        ],
        out_specs=pl.BlockSpec((br, DIM), lambda i: (i, 0)),
        out_shape=jax.ShapeDtypeStruct((N_NODES, DIM), jnp.float32),
    )(agg, degc, h, sw, gamma, beta)


def _tc_vq_stage(x, cb, cbt, n_actual, cb0row=None):
    n_pad = x.shape[0]
    s_sz = cb.shape[0]
    br = 256
    nb = n_pad // br

    def body(*refs):
        if cb0row is not None:
            x_ref, cb_ref, cbt_ref, c0_ref, o_ref, s_ref = refs
        else:
            x_ref, cb_ref, cbt_ref, o_ref, s_ref = refs
        i = pl.program_id(0)
        res = x_ref[...]
        if cb0row is not None:
            res = res - c0_ref[...]
        cbt_full = cbt_ref[...]
        nrm = jnp.sqrt(jnp.sum(cbt_full * cbt_full, axis=0, keepdims=True))
        cbnt = cbt_full / jnp.maximum(nrm, 1e-12)
        cn2 = jnp.sum(cbnt * cbnt, axis=0, keepdims=True)
        rn2 = jnp.sum(res * res, axis=-1, keepdims=True)
        d2 = rn2 + cn2 - 2.0 * jnp.dot(res, cbnt,
                                       preferred_element_type=jnp.float32)
        d2 = jnp.maximum(d2, 0.0)
        m = jnp.min(d2, axis=-1, keepdims=True)
        iota = lax.broadcasted_iota(jnp.int32, (br, s_sz), 1)
        sel = jnp.where(d2 == m, iota, s_sz)
        first = jnp.min(sel, axis=-1, keepdims=True)
        onehot = (iota == first).astype(jnp.float32)
        q = jnp.dot(onehot, cb_ref[...], preferred_element_type=jnp.float32)
        newres = res - q
        o_ref[...] = newres
        rowmask = (i * br + lax.broadcasted_iota(jnp.int32, (br, 1), 0)) < n_actual
        s_post = jnp.sum(jnp.where(rowmask, newres * newres, 0.0))
        if cb0row is not None:
            s_pre = jnp.sum(jnp.where(rowmask, res * res, 0.0))
        else:
            s_pre = s_post

        @pl.when(i == 0)
        def _():
            s_ref[...] = jnp.zeros_like(s_ref)

        r8 = lax.broadcasted_iota(jnp.int32, (8, 128), 0)
        c128 = lax.broadcasted_iota(jnp.int32, (8, 128), 1)
        add = jnp.where((r8 == 0) & (c128 == 0), s_pre, 0.0)
        add = add + jnp.where((r8 == 0) & (c128 == 1), s_post, 0.0)
        s_ref[...] = s_ref[...] + add

    in_specs = [
        pl.BlockSpec((br, DIM), lambda i: (i, 0)),
        pl.BlockSpec((s_sz, DIM), lambda i: (0, 0)),
        pl.BlockSpec((DIM, s_sz), lambda i: (0, 0)),
    ]
    args = [x, cb, cbt]
    if cb0row is not None:
        in_specs.append(pl.BlockSpec((1, DIM), lambda i: (0, 0)))
        args.append(cb0row)

    return _PC(
        body,
        grid=(nb,),
        in_specs=in_specs,
        out_specs=[
            pl.BlockSpec((br, DIM), lambda i: (i, 0)),
            pl.BlockSpec((8, 128), lambda i: (0, 0)),
        ],
        out_shape=[
            jax.ShapeDtypeStruct((n_pad, DIM), jnp.float32),
            jax.ShapeDtypeStruct((8, 128), jnp.float32),
        ],
    )(*args)


def _tc_score(hrows, trows, relw, reloh, gamma, beta):
    b = hrows.shape[0]

    def body(h_ref, t_ref, w_ref, r_ref, g_ref, b_ref, o_ref):
        def lnt(v):
            mu = jnp.mean(v, axis=-1, keepdims=True)
            var = jnp.mean((v - mu) ** 2, axis=-1, keepdims=True)
            return jnp.tanh((v - mu) / jnp.sqrt(var + 1e-5) * g_ref[...] + b_ref[...])

        hh = lnt(h_ref[...])
        ht = lnt(t_ref[...])
        score = jnp.zeros((b, 1), jnp.float32)
        for r in range(N_REL):
            cr = jnp.dot(hh, w_ref[r], preferred_element_type=jnp.float32)
            sr = jnp.sum(cr * ht, axis=-1, keepdims=True)
            score = score + sr * r_ref[:, r:r + 1]
        y = (lax.broadcasted_iota(jnp.int32, (b, 1), 0) < N_POS).astype(jnp.float32)
        e = (jnp.maximum(score, 0.0) - score * y
             + jnp.log(1.0 + jnp.exp(-jnp.abs(score))))
        total = jnp.sum(e) / b
        o_ref[...] = jnp.full((8, 128), total, jnp.float32)

    return _PC(
        body,
        grid=(1,),
        in_specs=[
            pl.BlockSpec((b, DIM), lambda i: (0, 0)),
            pl.BlockSpec((b, DIM), lambda i: (0, 0)),
            pl.BlockSpec((N_REL, DIM, DIM), lambda i: (0, 0, 0)),
            pl.BlockSpec((b, N_REL), lambda i: (0, 0)),
            pl.BlockSpec((1, DIM), lambda i: (0, 0)),
            pl.BlockSpec((1, DIM), lambda i: (0, 0)),
        ],
        out_specs=pl.BlockSpec((8, 128), lambda i: (0, 0)),
        out_shape=jax.ShapeDtypeStruct((8, 128), jnp.float32),
    )(hrows, trows, relw, reloh, gamma, beta)


_TYPE_RANGES = (("poi", 0, 4000), ("user", 4000, 7000),
                ("region", 7000, 9000), ("category", 9000, 10000))


def kernel(edge_index, edge_type, positive_triples, corrupted_tails, params):
    p = params
    src = edge_index[0]
    dst = edge_index[1]
    g = edge_type * N_NODES + src
    pad = EDGE_PAD - g.shape[0]
    g_pad = jnp.concatenate([g, jnp.zeros((pad,), jnp.int32)])
    dst_pad = jnp.concatenate([dst, jnp.full((pad,), -1, jnp.int32)])

    gamma = p["ln_gamma"][None, :]
    beta = p["ln_beta"][None, :]

    entries, counts = _sc_edge_scan(g_pad, dst_pad)

    h = p["node_emb"]
    degc = None
    for l in range(N_LAYERS):
        tr = _tc_rel_transform(h, p["rel_W"][l])
        if l == 0:
            agg, deg = _sc_edge_acc(tr, entries, counts, want_deg=True)
            degc = jnp.clip(deg, 1.0)[:, None]
        else:
            agg = _sc_edge_acc(tr, entries, counts, want_deg=False)
        h = _tc_layer_update(agg, degc, h, p["self_W"][l], gamma, beta,
                             final_ln=(l == N_LAYERS - 1))

    total_l_rq = jnp.float32(0.0)
    dec_parts = []
    for t, s0, e0 in _TYPE_RANGES:
        cbs = p["codebooks"][t]
        n = e0 - s0
        n_pad = ((n + 255) // 256) * 256
        x = h[s0:e0]
        xp = jnp.concatenate([x, jnp.zeros((n_pad - n, DIM), x.dtype)]) \
            if n_pad > n else x
        cb1, cb2 = cbs[1], cbs[2]
        r2, sums1 = _tc_vq_stage(xp, cb1, cb1.T, n, cb0row=cbs[0])
        r3, sums2 = _tc_vq_stage(r2, cb2, cb2.T, n, cb0row=None)
        denom = jnp.float32(n * DIM)
        total_l_rq = total_l_rq + 1.25 * (
            sums1[0, 0] + sums1[0, 1] + sums2[0, 1]) / denom
        dec_parts.append(x - r3[:n])
    dec = jnp.concatenate(dec_parts, axis=0)

    ph = positive_triples[:, 0]
    pr = positive_triples[:, 1]
    pt = positive_triples[:, 2]
    gidx = jnp.concatenate([ph, ph, pt, corrupted_tails])
    rows = _sc_gather_rows(dec, gidx)
    reloh = jax.nn.one_hot(jnp.concatenate([pr, pr]), N_REL, dtype=jnp.float32)
    sc_out = _tc_score(rows[:2 * N_POS], rows[2 * N_POS:],
                       p["relation_weights"], reloh, gamma, beta)
    return sc_out[0, 0] + total_l_rq


# speedup vs baseline: 1.7568x; 1.0804x over previous
import functools

import jax
import jax.numpy as jnp
from jax import lax
from jax.experimental import pallas as pl
from jax.experimental.pallas import tpu as pltpu
from jax.experimental.pallas import tpu_sc as plsc

N_NODES = 10000
N_REL = 8
DIM = 256
N_LAYERS = 3
N_POS = 1024

NC, NS = 2, 16
NW = NC * NS
CHUNK = 64
SCAN = 2048
EDGE_PAD = 80 * SCAN
NPASS = 2
BUCKET = 160
ACCROWS = 168
PDUMP = 160
NBKT = NW * NPASS
CAP = EDGE_PAD + 512
BLK = 512

_PC = pl.pallas_call


def _sc_edge_scan(g_pad, dst_pad):
    mesh = plsc.VectorSubcoreMesh(core_axis_name="c", subcore_axis_name="s")
    niter = EDGE_PAD // SCAN
    PEND = SCAN + 512

    @functools.partial(
        pl.kernel,
        out_type=[jax.ShapeDtypeStruct((NBKT * CAP,), jnp.int32),
                  jax.ShapeDtypeStruct((NBKT, 16), jnp.int32)],
        mesh=mesh,
        compiler_params=pltpu.CompilerParams(needs_layout_passes=False),
        scratch_types=[
            pltpu.VMEM((SCAN,), jnp.int32),
            pltpu.VMEM((SCAN,), jnp.int32),
            pltpu.VMEM((PEND,), jnp.int32),
            pltpu.VMEM((PEND,), jnp.int32),
            pltpu.VMEM((1, 16), jnp.int32),
        ],
    )
    def k(g_hbm, d_hbm, ent_hbm, cnt_hbm, gs_v, ds_v, pa, pb, cv):
        c = lax.axis_index("c")
        s = lax.axis_index("s")
        wid = s * NC + c
        iota16 = lax.iota(jnp.int32, 16)
        lo = wid * NPASS * BUCKET
        bktA = wid * NPASS
        bktB = bktA + 1

        def scan_iter(cc, carry):
            pcA, pcB, flA, flB = carry
            base = cc * SCAN
            pltpu.sync_copy(g_hbm.at[pl.ds(base, SCAN)], gs_v)
            pltpu.sync_copy(d_hbm.at[pl.ds(base, SCAN)], ds_v)

            def vec16(i, pc2):
                pcA2, pcB2 = pc2
                dvec = ds_v[pl.ds(i * 16, 16)]
                gvec = gs_v[pl.ds(i * 16, 16)]
                ldd = dvec - lo
                mA = (ldd >= 0) & (ldd < BUCKET)
                mB = (ldd >= BUCKET) & (ldd < 2 * BUCKET)
                pkA = gvec * 512 + jnp.where(mA, ldd, PDUMP)
                pkB = gvec * 512 + jnp.where(mB, ldd - BUCKET, PDUMP)
                plsc.store_compressed(pa.at[pl.ds(pcA2, 16)], pkA, mask=mA)
                plsc.store_compressed(pb.at[pl.ds(pcB2, 16)], pkB, mask=mB)
                cA = plsc.all_reduce_population_count(mA)
                cB = plsc.all_reduce_population_count(mB)
                return (pcA2 + cA[0], pcB2 + cB[0])

            pcA, pcB = lax.fori_loop(0, SCAN // 16, vec16, (pcA, pcB))

            def flush(pend, bkt, pc, fl):
                nb = pc // 512

                def fbody(j, carry):
                    off = pl.multiple_of(bkt * CAP + fl + j * 512, 512)
                    pltpu.sync_copy(pend.at[pl.ds(j * 512, 512)],
                                    ent_hbm.at[pl.ds(off, 512)])
                    return carry
                lax.fori_loop(0, nb, fbody, 0)
                rb = nb * 512
                for g16 in range(512 // 16):
                    pend[pl.ds(g16 * 16, 16)] = pend[pl.ds(rb + g16 * 16, 16)]
                return pc - rb, fl + rb

            pcA, flA = flush(pa, bktA, pcA, flA)
            pcB, flB = flush(pb, bktB, pcB, flB)
            return (pcA, pcB, flA, flB)

        pcA, pcB, flA, flB = lax.fori_loop(0, niter, scan_iter, (0, 0, 0, 0))

        def finish(pend, bkt, pc, fl):
            for g16 in range(512 // 16):
                lane = g16 * 16 + iota16
                pv = pend[pl.ds(g16 * 16, 16)]
                dumpg = wid * 2048 + lane * 4
                pend[pl.ds(g16 * 16, 16)] = jnp.where(
                    lane < pc, pv, dumpg * 512 + PDUMP)
            off = pl.multiple_of(bkt * CAP + fl, 512)
            pltpu.sync_copy(pend.at[pl.ds(0, 512)],
                            ent_hbm.at[pl.ds(off, 512)])
            tgt = jnp.maximum((pc + 127) // 128 * 128, 128)
            cv[0, pl.ds(0, 16)] = jnp.zeros((16,), jnp.int32) + (fl + tgt)
            pltpu.sync_copy(cv, cnt_hbm.at[pl.ds(bkt, 1)])

        finish(pa, bktA, pcA, flA)
        finish(pb, bktB, pcB, flB)

    return k(g_pad, dst_pad)


def _sc_edge_acc(tr_flat, entries, counts, want_deg):
    mesh = plsc.VectorSubcoreMesh(core_axis_name="c", subcore_axis_name="s")
    out_type = [jax.ShapeDtypeStruct((NBKT * BUCKET, DIM), jnp.float32)]
    if want_deg:
        out_type.append(jax.ShapeDtypeStruct((NBKT * BUCKET, 16), jnp.float32))

    @functools.partial(
        pl.kernel,
        out_type=out_type,
        mesh=mesh,
        compiler_params=pltpu.CompilerParams(needs_layout_passes=False),
        scratch_types=[
            pltpu.VMEM((512,), jnp.int32),
            pltpu.VMEM((CHUNK,), jnp.int32),
            pltpu.VMEM((CHUNK,), jnp.int32),
            pltpu.VMEM((CHUNK,), jnp.int32),
            pltpu.VMEM((CHUNK,), jnp.int32),
            pltpu.VMEM((CHUNK, DIM), jnp.float32),
            pltpu.VMEM((CHUNK, DIM), jnp.float32),
            pltpu.VMEM((ACCROWS, DIM), jnp.float32),
            pltpu.VMEM((ACCROWS, 16), jnp.float32),
            pltpu.VMEM((1, 16), jnp.int32),
            pltpu.SemaphoreType.DMA,
            pltpu.SemaphoreType.DMA,
        ],
    )
    def k(tr_hbm, ent_hbm, cnt_hbm, *rest):
        if want_deg:
            out_hbm, deg_hbm = rest[:2]
            rest = rest[2:]
        else:
            out_hbm, = rest[:1]
            rest = rest[1:]
        eblk, gi0, gi1, ld0, ld1, r0, r1, acc, dacc, cv, sem0, sem1 = rest
        gi = (gi0, gi1)
        ld = (ld0, ld1)
        rows = (r0, r1)
        sem = (sem0, sem1)
        c = lax.axis_index("c")
        s = lax.axis_index("s")
        wid = s * NC + c
        ones16 = jnp.ones((16,), jnp.float32)
        zeros16 = jnp.zeros((16,), jnp.float32)

        def issue(cidx, slot):
            off = lax.rem(cidx, 8) * CHUNK
            for jj in range(CHUNK // 16):
                pv = eblk[pl.ds(off + jj * 16, 16)]
                gi[slot][pl.ds(jj * 16, 16)] = lax.shift_right_logical(pv, 9)
                ld[slot][pl.ds(jj * 16, 16)] = lax.bitwise_and(pv, 511)
            pltpu.async_copy(tr_hbm.at[gi[slot]], rows[slot], sem[slot])

        def drain_acc(slot):
            pltpu.make_async_copy(tr_hbm.at[gi[slot]], rows[slot],
                                  sem[slot]).wait()

            def grp(g16, carry):
                ldvec = ld[slot][pl.ds(g16 * 16, 16)]
                for j2 in range(16):
                    lde = ldvec[j2]
                    for jj in range(DIM // 16):
                        xv = rows[slot][g16 * 16 + j2, pl.ds(jj * 16, 16)]
                        plsc.addupdate(acc.at[lde, pl.ds(jj * 16, 16)], xv)
                    if want_deg:
                        plsc.addupdate(dacc.at[lde, pl.ds(0, 16)], ones16)
                return carry
            lax.fori_loop(0, CHUNK // 16, grp, 0)

        def pass_body(p, carry):
            bkt = wid * NPASS + p
            lo = bkt * BUCKET

            def zrow(r, carry2):
                for j in range(DIM // 16):
                    acc[r, pl.ds(j * 16, 16)] = zeros16
                dacc[r, pl.ds(0, 16)] = zeros16
                return carry2
            lax.fori_loop(0, ACCROWS, zrow, 0)

            pltpu.sync_copy(cnt_hbm.at[pl.ds(bkt, 1)], cv)
            cvec = cv[0, pl.ds(0, 16)]
            nch = cvec[0] // CHUNK

            pltpu.sync_copy(
                ent_hbm.at[pl.ds(pl.multiple_of(bkt * CAP, 512), 512)], eblk)
            issue(0, 0)

            def pair(q, carry2):
                issue(2 * q + 1, 1)
                drain_acc(0)
                c2 = 2 * q + 2

                @pl.when(c2 < nch)
                def _():
                    @pl.when(lax.rem(c2, 8) == 0)
                    def _():
                        boff = pl.multiple_of(
                            bkt * CAP + (c2 // 8) * 512, 512)
                        pltpu.sync_copy(ent_hbm.at[pl.ds(boff, 512)], eblk)
                    issue(c2, 0)
                drain_acc(1)
                return carry2
            lax.fori_loop(0, nch // 2, pair, 0)

            pltpu.sync_copy(acc.at[pl.ds(0, BUCKET)],
                            out_hbm.at[pl.ds(lo, BUCKET)])
            if want_deg:
                pltpu.sync_copy(dacc.at[pl.ds(0, BUCKET)],
                                deg_hbm.at[pl.ds(lo, BUCKET)])
            return carry

        lax.fori_loop(0, NPASS, pass_body, 0)

    outs = k(tr_flat, entries, counts)
    agg = outs[0][:N_NODES]
    if want_deg:
        return agg, outs[1][:N_NODES, 0]
    return agg


def _sc_gather_rows(table, idx):
    b = idx.shape[0]
    bpw = b // (NC * NS)
    mesh = plsc.VectorSubcoreMesh(core_axis_name="c", subcore_axis_name="s")

    @functools.partial(
        pl.kernel,
        out_type=jax.ShapeDtypeStruct((b, DIM), jnp.float32),
        mesh=mesh,
        compiler_params=pltpu.CompilerParams(needs_layout_passes=False),
        scratch_types=[
            pltpu.VMEM((bpw,), jnp.int32),
            pltpu.VMEM((bpw, DIM), jnp.float32),
            pltpu.SemaphoreType.DMA,
        ],
    )
    def k(tab_hbm, idx_hbm, out_hbm, idx_v, rows_v, sem):
        wid = lax.axis_index("s") * NC + lax.axis_index("c")
        base = wid * bpw
        pltpu.sync_copy(idx_hbm.at[pl.ds(base, bpw)], idx_v)
        pltpu.async_copy(tab_hbm.at[idx_v], rows_v, sem).wait()
        pltpu.sync_copy(rows_v, out_hbm.at[pl.ds(base, bpw)])

    return k(table, idx)


def _tc_rel_transform(h, w):
    br = 2000
    nb = N_NODES // br

    def body(h_ref, w_ref, o_ref):
        o_ref[...] = jnp.dot(h_ref[...], w_ref[0],
                             preferred_element_type=jnp.float32)

    return _PC(
        body,
        grid=(N_REL, nb),
        in_specs=[
            pl.BlockSpec((br, DIM), lambda r, i: (i, 0)),
            pl.BlockSpec((1, DIM, DIM), lambda r, i: (r, 0, 0)),
        ],
        out_specs=pl.BlockSpec((br, DIM), lambda r, i: (r * nb + i, 0)),
        out_shape=jax.ShapeDtypeStruct((N_REL * N_NODES, DIM), jnp.float32),
    )(h, w)


def _tc_layer_update(agg, degc, h, sw, gamma, beta, final_ln):
    br = 2000

    def body(a_ref, d_ref, h_ref, w_ref, g_ref, b_ref, o_ref):
        hn = a_ref[...] / d_ref[...] + jnp.dot(
            h_ref[...], w_ref[...], preferred_element_type=jnp.float32)
        hn = jnp.maximum(hn, 0.0)
        if final_ln:
            mu = jnp.mean(hn, axis=-1, keepdims=True)
            var = jnp.mean((hn - mu) ** 2, axis=-1, keepdims=True)
            hn = (hn - mu) / jnp.sqrt(var + 1e-5) * g_ref[...] + b_ref[...]
        o_ref[...] = hn

    return _PC(
        body,
        grid=(N_NODES // br,),
        in_specs=[
            pl.BlockSpec((br, DIM), lambda i: (i, 0)),
            pl.BlockSpec((br, 1), lambda i: (i, 0)),
            pl.BlockSpec((br, DIM), lambda i: (i, 0)),
            pl.BlockSpec((DIM, DIM), lambda i: (0, 0)),
            pl.BlockSpec((1, DIM), lambda i: (0, 0)),
            pl.BlockSpec((1, DIM), lambda i: (0, 0)),
        ],
        out_specs=pl.BlockSpec((br, DIM), lambda i: (i, 0)),
        out_shape=jax.ShapeDtypeStruct((N_NODES, DIM), jnp.float32),
    )(agg, degc, h, sw, gamma, beta)


def _tc_vq_type(x, cb0row, cb1, cb2, n_actual):
    n_pad = x.shape[0]
    br = 256
    nb = n_pad // br

    def stage(res, cb_ref, cbt_ref, s_sz):
        cbt_full = cbt_ref[...]
        nrm = jnp.sqrt(jnp.sum(cbt_full * cbt_full, axis=0, keepdims=True))
        cbnt = cbt_full / jnp.maximum(nrm, 1e-12)
        cn2 = jnp.sum(cbnt * cbnt, axis=0, keepdims=True)
        rn2 = jnp.sum(res * res, axis=-1, keepdims=True)
        d2 = rn2 + cn2 - 2.0 * jnp.dot(res, cbnt,
                                       preferred_element_type=jnp.float32)
        d2 = jnp.maximum(d2, 0.0)
        m = jnp.min(d2, axis=-1, keepdims=True)
        iota = lax.broadcasted_iota(jnp.int32, (br, s_sz), 1)
        sel = jnp.where(d2 == m, iota, s_sz)
        first = jnp.min(sel, axis=-1, keepdims=True)
        onehot = (iota == first).astype(jnp.float32)
        q = jnp.dot(onehot, cb_ref[...], preferred_element_type=jnp.float32)
        return res - q

    def body(x_ref, c0_ref, cb1_ref, cb1t_ref, cb2_ref, cb2t_ref,
             o_ref, s_ref):
        i = pl.program_id(0)
        xv = x_ref[...]
        r1 = xv - c0_ref[...]
        r2 = stage(r1, cb1_ref, cb1t_ref, cb1.shape[0])
        r3 = stage(r2, cb2_ref, cb2t_ref, cb2.shape[0])
        o_ref[...] = xv - r3
        rowmask = (i * br + lax.broadcasted_iota(jnp.int32, (br, 1), 0)) < n_actual
        s1 = jnp.sum(jnp.where(rowmask, r1 * r1, 0.0))
        s2 = jnp.sum(jnp.where(rowmask, r2 * r2, 0.0))
        s3 = jnp.sum(jnp.where(rowmask, r3 * r3, 0.0))

        @pl.when(i == 0)
        def _():
            s_ref[...] = jnp.zeros_like(s_ref)

        r8 = lax.broadcasted_iota(jnp.int32, (8, 128), 0)
        c128 = lax.broadcasted_iota(jnp.int32, (8, 128), 1)
        add = (jnp.where((r8 == 0) & (c128 == 0), s1, 0.0)
               + jnp.where((r8 == 0) & (c128 == 1), s2, 0.0)
               + jnp.where((r8 == 0) & (c128 == 2), s3, 0.0))
        s_ref[...] = s_ref[...] + add

    return _PC(
        body,
        grid=(nb,),
        in_specs=[
            pl.BlockSpec((br, DIM), lambda i: (i, 0)),
            pl.BlockSpec((1, DIM), lambda i: (0, 0)),
            pl.BlockSpec((cb1.shape[0], DIM), lambda i: (0, 0)),
            pl.BlockSpec((DIM, cb1.shape[0]), lambda i: (0, 0)),
            pl.BlockSpec((cb2.shape[0], DIM), lambda i: (0, 0)),
            pl.BlockSpec((DIM, cb2.shape[0]), lambda i: (0, 0)),
        ],
        out_specs=[
            pl.BlockSpec((br, DIM), lambda i: (i, 0)),
            pl.BlockSpec((8, 128), lambda i: (0, 0)),
        ],
        out_shape=[
            jax.ShapeDtypeStruct((n_pad, DIM), jnp.float32),
            jax.ShapeDtypeStruct((8, 128), jnp.float32),
        ],
    )(x, cb0row, cb1, cb1.T, cb2, cb2.T)


def _tc_score(hrows, trows, relw, reloh, gamma, beta):
    b = hrows.shape[0]

    def body(h_ref, t_ref, w_ref, r_ref, g_ref, b_ref, o_ref):
        def lnt(v):
            mu = jnp.mean(v, axis=-1, keepdims=True)
            var = jnp.mean((v - mu) ** 2, axis=-1, keepdims=True)
            return jnp.tanh((v - mu) / jnp.sqrt(var + 1e-5) * g_ref[...] + b_ref[...])

        hh = lnt(h_ref[...])
        ht = lnt(t_ref[...])
        score = jnp.zeros((b, 1), jnp.float32)
        for r in range(N_REL):
            cr = jnp.dot(hh, w_ref[r], preferred_element_type=jnp.float32)
            sr = jnp.sum(cr * ht, axis=-1, keepdims=True)
            score = score + sr * r_ref[:, r:r + 1]
        y = (lax.broadcasted_iota(jnp.int32, (b, 1), 0) < N_POS).astype(jnp.float32)
        e = (jnp.maximum(score, 0.0) - score * y
             + jnp.log(1.0 + jnp.exp(-jnp.abs(score))))
        total = jnp.sum(e) / b
        o_ref[...] = jnp.full((8, 128), total, jnp.float32)

    return _PC(
        body,
        grid=(1,),
        in_specs=[
            pl.BlockSpec((b, DIM), lambda i: (0, 0)),
            pl.BlockSpec((b, DIM), lambda i: (0, 0)),
            pl.BlockSpec((N_REL, DIM, DIM), lambda i: (0, 0, 0)),
            pl.BlockSpec((b, N_REL), lambda i: (0, 0)),
            pl.BlockSpec((1, DIM), lambda i: (0, 0)),
            pl.BlockSpec((1, DIM), lambda i: (0, 0)),
        ],
        out_specs=pl.BlockSpec((8, 128), lambda i: (0, 0)),
        out_shape=jax.ShapeDtypeStruct((8, 128), jnp.float32),
    )(hrows, trows, relw, reloh, gamma, beta)


_TYPE_RANGES = (("poi", 0, 4000), ("user", 4000, 7000),
                ("region", 7000, 9000), ("category", 9000, 10000))


def kernel(edge_index, edge_type, positive_triples, corrupted_tails, params):
    p = params
    src = edge_index[0]
    dst = edge_index[1]
    g = edge_type * N_NODES + src
    pad = EDGE_PAD - g.shape[0]
    g_pad = jnp.concatenate([g, jnp.zeros((pad,), jnp.int32)])
    dst_pad = jnp.concatenate([dst, jnp.full((pad,), -1, jnp.int32)])

    gamma = p["ln_gamma"][None, :]
    beta = p["ln_beta"][None, :]

    entries, counts = _sc_edge_scan(g_pad, dst_pad)

    h = p["node_emb"]
    degc = None
    for l in range(N_LAYERS):
        tr = _tc_rel_transform(h, p["rel_W"][l])
        if l == 0:
            agg, deg = _sc_edge_acc(tr, entries, counts, want_deg=True)
            degc = jnp.clip(deg, 1.0)[:, None]
        else:
            agg = _sc_edge_acc(tr, entries, counts, want_deg=False)
        h = _tc_layer_update(agg, degc, h, p["self_W"][l], gamma, beta,
                             final_ln=(l == N_LAYERS - 1))

    total_l_rq = jnp.float32(0.0)
    dec_parts = []
    for t, s0, e0 in _TYPE_RANGES:
        cbs = p["codebooks"][t]
        n = e0 - s0
        n_pad = ((n + 255) // 256) * 256
        x = h[s0:e0]
        xp = jnp.concatenate([x, jnp.zeros((n_pad - n, DIM), x.dtype)]) \
            if n_pad > n else x
        dec, sums = _tc_vq_type(xp, cbs[0], cbs[1], cbs[2], n)
        denom = jnp.float32(n * DIM)
        total_l_rq = total_l_rq + 1.25 * (
            sums[0, 0] + sums[0, 1] + sums[0, 2]) / denom
        dec_parts.append(dec[:n])
    dec = jnp.concatenate(dec_parts, axis=0)

    ph = positive_triples[:, 0]
    pr = positive_triples[:, 1]
    pt = positive_triples[:, 2]
    gidx = jnp.concatenate([ph, ph, pt, corrupted_tails])
    rows = _sc_gather_rows(dec, gidx)
    reloh = jax.nn.one_hot(jnp.concatenate([pr, pr]), N_REL, dtype=jnp.float32)
    sc_out = _tc_score(rows[:2 * N_POS], rows[2 * N_POS:],
                       p["relation_weights"], reloh, gamma, beta)
    return sc_out[0, 0] + total_l_rq


# speedup vs baseline: 1.8594x; 1.0584x over previous
import functools

import jax
import jax.numpy as jnp
from jax import lax
from jax.experimental import pallas as pl
from jax.experimental.pallas import tpu as pltpu
from jax.experimental.pallas import tpu_sc as plsc

N_NODES = 10000
N_REL = 8
DIM = 256
N_LAYERS = 3
N_POS = 1024

NC, NS = 2, 16
NW = NC * NS
CHUNK = 64
SCAN = 2048
EDGE_PAD = 80 * SCAN
NPASS = 2
BUCKET = 160
ACCROWS = 168
PDUMP = 160
NBKT = NW * NPASS
CAP = EDGE_PAD + 512
BLK = 512

_PC = pl.pallas_call


def _sc_edge_scan(g_pad, dst_pad):
    mesh = plsc.VectorSubcoreMesh(core_axis_name="c", subcore_axis_name="s")
    niter = EDGE_PAD // SCAN
    PEND = SCAN + 512

    @functools.partial(
        pl.kernel,
        out_type=[jax.ShapeDtypeStruct((NBKT * CAP,), jnp.int32),
                  jax.ShapeDtypeStruct((NBKT, 16), jnp.int32)],
        mesh=mesh,
        compiler_params=pltpu.CompilerParams(needs_layout_passes=False),
        scratch_types=[
            pltpu.VMEM((SCAN,), jnp.int32),
            pltpu.VMEM((SCAN,), jnp.int32),
            pltpu.VMEM((PEND,), jnp.int32),
            pltpu.VMEM((PEND,), jnp.int32),
            pltpu.VMEM((1, 16), jnp.int32),
        ],
    )
    def k(g_hbm, d_hbm, ent_hbm, cnt_hbm, gs_v, ds_v, pa, pb, cv):
        c = lax.axis_index("c")
        s = lax.axis_index("s")
        wid = s * NC + c
        iota16 = lax.iota(jnp.int32, 16)
        lo = wid * NPASS * BUCKET
        bktA = wid * NPASS
        bktB = bktA + 1

        def scan_iter(cc, carry):
            pcA, pcB, flA, flB = carry
            base = cc * SCAN
            pltpu.sync_copy(g_hbm.at[pl.ds(base, SCAN)], gs_v)
            pltpu.sync_copy(d_hbm.at[pl.ds(base, SCAN)], ds_v)

            def vec16(i, pc2):
                pcA2, pcB2 = pc2
                dvec = ds_v[pl.ds(i * 16, 16)]
                gvec = gs_v[pl.ds(i * 16, 16)]
                ldd = dvec - lo
                mA = (ldd >= 0) & (ldd < BUCKET)
                mB = (ldd >= BUCKET) & (ldd < 2 * BUCKET)
                pkA = gvec * 512 + jnp.where(mA, ldd, PDUMP)
                pkB = gvec * 512 + jnp.where(mB, ldd - BUCKET, PDUMP)
                plsc.store_compressed(pa.at[pl.ds(pcA2, 16)], pkA, mask=mA)
                plsc.store_compressed(pb.at[pl.ds(pcB2, 16)], pkB, mask=mB)
                cA = plsc.all_reduce_population_count(mA)
                cB = plsc.all_reduce_population_count(mB)
                return (pcA2 + cA[0], pcB2 + cB[0])

            pcA, pcB = lax.fori_loop(0, SCAN // 16, vec16, (pcA, pcB))

            def flush(pend, bkt, pc, fl):
                nb = pc // 512

                def fbody(j, carry):
                    off = pl.multiple_of(bkt * CAP + fl + j * 512, 512)
                    pltpu.sync_copy(pend.at[pl.ds(j * 512, 512)],
                                    ent_hbm.at[pl.ds(off, 512)])
                    return carry
                lax.fori_loop(0, nb, fbody, 0)
                rb = nb * 512
                for g16 in range(512 // 16):
                    pend[pl.ds(g16 * 16, 16)] = pend[pl.ds(rb + g16 * 16, 16)]
                return pc - rb, fl + rb

            pcA, flA = flush(pa, bktA, pcA, flA)
            pcB, flB = flush(pb, bktB, pcB, flB)
            return (pcA, pcB, flA, flB)

        pcA, pcB, flA, flB = lax.fori_loop(0, niter, scan_iter, (0, 0, 0, 0))

        def finish(pend, bkt, pc, fl):
            for g16 in range(512 // 16):
                lane = g16 * 16 + iota16
                pv = pend[pl.ds(g16 * 16, 16)]
                dumpg = wid * 2048 + lane * 4
                pend[pl.ds(g16 * 16, 16)] = jnp.where(
                    lane < pc, pv, dumpg * 512 + PDUMP)
            off = pl.multiple_of(bkt * CAP + fl, 512)
            pltpu.sync_copy(pend.at[pl.ds(0, 512)],
                            ent_hbm.at[pl.ds(off, 512)])
            tgt = jnp.maximum((pc + 127) // 128 * 128, 128)
            cv[0, pl.ds(0, 16)] = jnp.zeros((16,), jnp.int32) + (fl + tgt)
            pltpu.sync_copy(cv, cnt_hbm.at[pl.ds(bkt, 1)])

        finish(pa, bktA, pcA, flA)
        finish(pb, bktB, pcB, flB)

    return k(g_pad, dst_pad)


def _sc_edge_acc(tr_flat, entries, counts, want_deg):
    mesh = plsc.VectorSubcoreMesh(core_axis_name="c", subcore_axis_name="s")
    out_type = [jax.ShapeDtypeStruct((NBKT * BUCKET, DIM), jnp.float32)]
    if want_deg:
        out_type.append(jax.ShapeDtypeStruct((NBKT * BUCKET, 16), jnp.float32))

    @functools.partial(
        pl.kernel,
        out_type=out_type,
        mesh=mesh,
        compiler_params=pltpu.CompilerParams(needs_layout_passes=False),
        scratch_types=[
            pltpu.VMEM((512,), jnp.int32),
            pltpu.VMEM((CHUNK,), jnp.int32),
            pltpu.VMEM((CHUNK,), jnp.int32),
            pltpu.VMEM((CHUNK,), jnp.int32),
            pltpu.VMEM((CHUNK,), jnp.int32),
            pltpu.VMEM((CHUNK, DIM), jnp.float32),
            pltpu.VMEM((CHUNK, DIM), jnp.float32),
            pltpu.VMEM((ACCROWS, DIM), jnp.float32),
            pltpu.VMEM((ACCROWS, 16), jnp.float32),
            pltpu.VMEM((1, 16), jnp.int32),
            pltpu.SemaphoreType.DMA,
            pltpu.SemaphoreType.DMA,
        ],
    )
    def k(tr_hbm, ent_hbm, cnt_hbm, *rest):
        if want_deg:
            out_hbm, deg_hbm = rest[:2]
            rest = rest[2:]
        else:
            out_hbm, = rest[:1]
            rest = rest[1:]
        eblk, gi0, gi1, ld0, ld1, r0, r1, acc, dacc, cv, sem0, sem1 = rest
        gi = (gi0, gi1)
        ld = (ld0, ld1)
        rows = (r0, r1)
        sem = (sem0, sem1)
        c = lax.axis_index("c")
        s = lax.axis_index("s")
        wid = s * NC + c
        ones16 = jnp.ones((16,), jnp.float32)
        zeros16 = jnp.zeros((16,), jnp.float32)

        def issue(cidx, slot):
            off = lax.rem(cidx, 8) * CHUNK
            for jj in range(CHUNK // 16):
                pv = eblk[pl.ds(off + jj * 16, 16)]
                gi[slot][pl.ds(jj * 16, 16)] = lax.shift_right_logical(pv, 9)
                ld[slot][pl.ds(jj * 16, 16)] = lax.bitwise_and(pv, 511)
            pltpu.async_copy(tr_hbm.at[gi[slot]], rows[slot], sem[slot])

        def drain_acc(slot):
            pltpu.make_async_copy(tr_hbm.at[gi[slot]], rows[slot],
                                  sem[slot]).wait()

            def grp(g16, carry):
                ldvec = ld[slot][pl.ds(g16 * 16, 16)]
                for j2 in range(16):
                    lde = ldvec[j2]
                    for jj in range(DIM // 16):
                        xv = rows[slot][g16 * 16 + j2, pl.ds(jj * 16, 16)]
                        plsc.addupdate(acc.at[lde, pl.ds(jj * 16, 16)], xv)
                    if want_deg:
                        plsc.addupdate(dacc.at[lde, pl.ds(0, 16)], ones16)
                return carry
            lax.fori_loop(0, CHUNK // 16, grp, 0)

        def pass_body(p, carry):
            bkt = wid * NPASS + p
            lo = bkt * BUCKET

            def zrow(r, carry2):
                for j in range(DIM // 16):
                    acc[r, pl.ds(j * 16, 16)] = zeros16
                dacc[r, pl.ds(0, 16)] = zeros16
                return carry2
            lax.fori_loop(0, ACCROWS, zrow, 0)

            pltpu.sync_copy(cnt_hbm.at[pl.ds(bkt, 1)], cv)
            cvec = cv[0, pl.ds(0, 16)]
            nch = cvec[0] // CHUNK

            pltpu.sync_copy(
                ent_hbm.at[pl.ds(pl.multiple_of(bkt * CAP, 512), 512)], eblk)
            issue(0, 0)

            def pair(q, carry2):
                issue(2 * q + 1, 1)
                drain_acc(0)
                c2 = 2 * q + 2

                @pl.when(c2 < nch)
                def _():
                    @pl.when(lax.rem(c2, 8) == 0)
                    def _():
                        boff = pl.multiple_of(
                            bkt * CAP + (c2 // 8) * 512, 512)
                        pltpu.sync_copy(ent_hbm.at[pl.ds(boff, 512)], eblk)
                    issue(c2, 0)
                drain_acc(1)
                return carry2
            lax.fori_loop(0, nch // 2, pair, 0)

            pltpu.sync_copy(acc.at[pl.ds(0, BUCKET)],
                            out_hbm.at[pl.ds(lo, BUCKET)])
            if want_deg:
                pltpu.sync_copy(dacc.at[pl.ds(0, BUCKET)],
                                deg_hbm.at[pl.ds(lo, BUCKET)])
            return carry

        lax.fori_loop(0, NPASS, pass_body, 0)

    return k(tr_flat, entries, counts)


def _sc_gather_rows(table, idx):
    b = idx.shape[0]
    bpw = b // (NC * NS)
    mesh = plsc.VectorSubcoreMesh(core_axis_name="c", subcore_axis_name="s")

    @functools.partial(
        pl.kernel,
        out_type=jax.ShapeDtypeStruct((b, DIM), jnp.float32),
        mesh=mesh,
        compiler_params=pltpu.CompilerParams(needs_layout_passes=False),
        scratch_types=[
            pltpu.VMEM((bpw,), jnp.int32),
            pltpu.VMEM((bpw, DIM), jnp.float32),
            pltpu.SemaphoreType.DMA,
        ],
    )
    def k(tab_hbm, idx_hbm, out_hbm, idx_v, rows_v, sem):
        wid = lax.axis_index("s") * NC + lax.axis_index("c")
        base = wid * bpw
        pltpu.sync_copy(idx_hbm.at[pl.ds(base, bpw)], idx_v)
        pltpu.async_copy(tab_hbm.at[idx_v], rows_v, sem).wait()
        pltpu.sync_copy(rows_v, out_hbm.at[pl.ds(base, bpw)])

    return k(table, idx)


def _tc_rel_transform(h, w):
    br = 2000
    nb = N_NODES // br

    def body(h_ref, w_ref, o_ref):
        o_ref[...] = jnp.dot(h_ref[...], w_ref[0],
                             preferred_element_type=jnp.float32)

    return _PC(
        body,
        grid=(N_REL, nb),
        in_specs=[
            pl.BlockSpec((br, DIM), lambda r, i: (i, 0)),
            pl.BlockSpec((1, DIM, DIM), lambda r, i: (r, 0, 0)),
        ],
        out_specs=pl.BlockSpec((br, DIM), lambda r, i: (r * nb + i, 0)),
        out_shape=jax.ShapeDtypeStruct((N_REL * N_NODES, DIM), jnp.float32),
    )(h, w)


def _tc_layer_update(agg_pad, deg_pad, h, sw, gamma, beta, final_ln):
    br = 2000

    def body(a_ref, d_ref, h_ref, w_ref, g_ref, b_ref, o_ref):
        dv = jnp.maximum(d_ref[:, 0:1], 1.0)
        hn = a_ref[...] / dv + jnp.dot(
            h_ref[...], w_ref[...], preferred_element_type=jnp.float32)
        hn = jnp.maximum(hn, 0.0)
        if final_ln:
            mu = jnp.mean(hn, axis=-1, keepdims=True)
            var = jnp.mean((hn - mu) ** 2, axis=-1, keepdims=True)
            hn = (hn - mu) / jnp.sqrt(var + 1e-5) * g_ref[...] + b_ref[...]
        o_ref[...] = hn

    return _PC(
        body,
        grid=(N_NODES // br,),
        in_specs=[
            pl.BlockSpec((br, DIM), lambda i: (i, 0)),
            pl.BlockSpec((br, 16), lambda i: (i, 0)),
            pl.BlockSpec((br, DIM), lambda i: (i, 0)),
            pl.BlockSpec((DIM, DIM), lambda i: (0, 0)),
            pl.BlockSpec((1, DIM), lambda i: (0, 0)),
            pl.BlockSpec((1, DIM), lambda i: (0, 0)),
        ],
        out_specs=pl.BlockSpec((br, DIM), lambda i: (i, 0)),
        out_shape=jax.ShapeDtypeStruct((N_NODES, DIM), jnp.float32),
    )(agg_pad, deg_pad, h, sw, gamma, beta)


def _tc_update_transform(agg_pad, deg_pad, h, sw, wnext):
    br = 2000

    def body(a_ref, d_ref, h_ref, w_ref, wn_ref, o_ref, t_ref):
        dv = jnp.maximum(d_ref[:, 0:1], 1.0)
        hn = a_ref[...] / dv + jnp.dot(
            h_ref[...], w_ref[...], preferred_element_type=jnp.float32)
        hn = jnp.maximum(hn, 0.0)
        o_ref[...] = hn
        for r in range(N_REL):
            t_ref[r] = jnp.dot(hn, wn_ref[r],
                               preferred_element_type=jnp.float32)

    return _PC(
        body,
        grid=(N_NODES // br,),
        in_specs=[
            pl.BlockSpec((br, DIM), lambda i: (i, 0)),
            pl.BlockSpec((br, 16), lambda i: (i, 0)),
            pl.BlockSpec((br, DIM), lambda i: (i, 0)),
            pl.BlockSpec((DIM, DIM), lambda i: (0, 0)),
            pl.BlockSpec((N_REL, DIM, DIM), lambda i: (0, 0, 0)),
        ],
        out_specs=[
            pl.BlockSpec((br, DIM), lambda i: (i, 0)),
            pl.BlockSpec((N_REL, br, DIM), lambda i: (0, i, 0)),
        ],
        out_shape=[
            jax.ShapeDtypeStruct((N_NODES, DIM), jnp.float32),
            jax.ShapeDtypeStruct((N_REL, N_NODES, DIM), jnp.float32),
        ],
    )(agg_pad, deg_pad, h, sw, wnext)


def _tc_vq_type(x, cb0row, cb1, cb2, n_actual):
    n_pad = x.shape[0]
    br = 256
    nb = n_pad // br

    def stage(res, cb_ref, cbt_ref, s_sz):
        cbt_full = cbt_ref[...]
        nrm = jnp.sqrt(jnp.sum(cbt_full * cbt_full, axis=0, keepdims=True))
        cbnt = cbt_full / jnp.maximum(nrm, 1e-12)
        cn2 = jnp.sum(cbnt * cbnt, axis=0, keepdims=True)
        rn2 = jnp.sum(res * res, axis=-1, keepdims=True)
        d2 = rn2 + cn2 - 2.0 * jnp.dot(res, cbnt,
                                       preferred_element_type=jnp.float32)
        d2 = jnp.maximum(d2, 0.0)
        m = jnp.min(d2, axis=-1, keepdims=True)
        iota = lax.broadcasted_iota(jnp.int32, (br, s_sz), 1)
        sel = jnp.where(d2 == m, iota, s_sz)
        first = jnp.min(sel, axis=-1, keepdims=True)
        onehot = (iota == first).astype(jnp.float32)
        q = jnp.dot(onehot, cb_ref[...], preferred_element_type=jnp.float32)
        return res - q

    def body(x_ref, c0_ref, cb1_ref, cb1t_ref, cb2_ref, cb2t_ref,
             o_ref, s_ref):
        i = pl.program_id(0)
        xv = x_ref[...]
        r1 = xv - c0_ref[...]
        r2 = stage(r1, cb1_ref, cb1t_ref, cb1.shape[0])
        r3 = stage(r2, cb2_ref, cb2t_ref, cb2.shape[0])
        o_ref[...] = xv - r3
        rowmask = (i * br + lax.broadcasted_iota(jnp.int32, (br, 1), 0)) < n_actual
        s1 = jnp.sum(jnp.where(rowmask, r1 * r1, 0.0))
        s2 = jnp.sum(jnp.where(rowmask, r2 * r2, 0.0))
        s3 = jnp.sum(jnp.where(rowmask, r3 * r3, 0.0))

        @pl.when(i == 0)
        def _():
            s_ref[...] = jnp.zeros_like(s_ref)

        r8 = lax.broadcasted_iota(jnp.int32, (8, 128), 0)
        c128 = lax.broadcasted_iota(jnp.int32, (8, 128), 1)
        add = (jnp.where((r8 == 0) & (c128 == 0), s1, 0.0)
               + jnp.where((r8 == 0) & (c128 == 1), s2, 0.0)
               + jnp.where((r8 == 0) & (c128 == 2), s3, 0.0))
        s_ref[...] = s_ref[...] + add

    return _PC(
        body,
        grid=(nb,),
        in_specs=[
            pl.BlockSpec((br, DIM), lambda i: (i, 0)),
            pl.BlockSpec((1, DIM), lambda i: (0, 0)),
            pl.BlockSpec((cb1.shape[0], DIM), lambda i: (0, 0)),
            pl.BlockSpec((DIM, cb1.shape[0]), lambda i: (0, 0)),
            pl.BlockSpec((cb2.shape[0], DIM), lambda i: (0, 0)),
            pl.BlockSpec((DIM, cb2.shape[0]), lambda i: (0, 0)),
        ],
        out_specs=[
            pl.BlockSpec((br, DIM), lambda i: (i, 0)),
            pl.BlockSpec((8, 128), lambda i: (0, 0)),
        ],
        out_shape=[
            jax.ShapeDtypeStruct((n_pad, DIM), jnp.float32),
            jax.ShapeDtypeStruct((8, 128), jnp.float32),
        ],
    )(x, cb0row, cb1, cb1.T, cb2, cb2.T)


def _tc_score(hrows, trows, relw, prr, gamma, beta):
    b = hrows.shape[0]

    def body(h_ref, t_ref, w_ref, r_ref, g_ref, b_ref, o_ref):
        def lnt(v):
            mu = jnp.mean(v, axis=-1, keepdims=True)
            var = jnp.mean((v - mu) ** 2, axis=-1, keepdims=True)
            return jnp.tanh((v - mu) / jnp.sqrt(var + 1e-5) * g_ref[...] + b_ref[...])

        hh = lnt(h_ref[...])
        ht = lnt(t_ref[...])
        rid = r_ref[...]
        score = jnp.zeros((b, 1), jnp.float32)
        for r in range(N_REL):
            cr = jnp.dot(hh, w_ref[r], preferred_element_type=jnp.float32)
            sr = jnp.sum(cr * ht, axis=-1, keepdims=True)
            score = score + sr * (rid == r).astype(jnp.float32)
        y = (lax.broadcasted_iota(jnp.int32, (b, 1), 0) < N_POS).astype(jnp.float32)
        e = (jnp.maximum(score, 0.0) - score * y
             + jnp.log(1.0 + jnp.exp(-jnp.abs(score))))
        total = jnp.sum(e) / b
        o_ref[...] = jnp.full((8, 128), total, jnp.float32)

    return _PC(
        body,
        grid=(1,),
        in_specs=[
            pl.BlockSpec((b, DIM), lambda i: (0, 0)),
            pl.BlockSpec((b, DIM), lambda i: (0, 0)),
            pl.BlockSpec((N_REL, DIM, DIM), lambda i: (0, 0, 0)),
            pl.BlockSpec((b, 1), lambda i: (0, 0)),
            pl.BlockSpec((1, DIM), lambda i: (0, 0)),
            pl.BlockSpec((1, DIM), lambda i: (0, 0)),
        ],
        out_specs=pl.BlockSpec((8, 128), lambda i: (0, 0)),
        out_shape=jax.ShapeDtypeStruct((8, 128), jnp.float32),
    )(hrows, trows, relw, prr, gamma, beta)


_TYPE_RANGES = (("poi", 0, 4000), ("user", 4000, 7000),
                ("region", 7000, 9000), ("category", 9000, 10000))


def kernel(edge_index, edge_type, positive_triples, corrupted_tails, params):
    p = params
    src = edge_index[0]
    dst = edge_index[1]
    g = edge_type * N_NODES + src
    pad = EDGE_PAD - g.shape[0]
    g_pad = jnp.concatenate([g, jnp.zeros((pad,), jnp.int32)])
    dst_pad = jnp.concatenate([dst, jnp.full((pad,), -1, jnp.int32)])

    gamma = p["ln_gamma"][None, :]
    beta = p["ln_beta"][None, :]

    entries, counts = _sc_edge_scan(g_pad, dst_pad)

    h = p["node_emb"]
    tr = _tc_rel_transform(h, p["rel_W"][0])
    agg_pad, deg_pad = _sc_edge_acc(tr, entries, counts, want_deg=True)
    for l in range(N_LAYERS - 1):
        h, tr3 = _tc_update_transform(agg_pad, deg_pad, h, p["self_W"][l],
                                      p["rel_W"][l + 1])
        agg_pad, = _sc_edge_acc(tr3.reshape(N_REL * N_NODES, DIM),
                                entries, counts, want_deg=False)
    h = _tc_layer_update(agg_pad, deg_pad, h, p["self_W"][N_LAYERS - 1],
                         gamma, beta, final_ln=True)

    total_l_rq = jnp.float32(0.0)
    dec_parts = []
    for t, s0, e0 in _TYPE_RANGES:
        cbs = p["codebooks"][t]
        n = e0 - s0
        n_pad = ((n + 255) // 256) * 256
        x = h[s0:e0]
        xp = jnp.concatenate([x, jnp.zeros((n_pad - n, DIM), x.dtype)]) \
            if n_pad > n else x
        dec, sums = _tc_vq_type(xp, cbs[0], cbs[1], cbs[2], n)
        denom = jnp.float32(n * DIM)
        total_l_rq = total_l_rq + 1.25 * (
            sums[0, 0] + sums[0, 1] + sums[0, 2]) / denom
        dec_parts.append(dec[:n])
    dec = jnp.concatenate(dec_parts, axis=0)

    ph = positive_triples[:, 0]
    pr = positive_triples[:, 1]
    pt = positive_triples[:, 2]
    gidx = jnp.concatenate([ph, ph, pt, corrupted_tails])
    rows = _sc_gather_rows(dec, gidx)
    prr = jnp.concatenate([pr, pr])[:, None]
    sc_out = _tc_score(rows[:2 * N_POS], rows[2 * N_POS:],
                       p["relation_weights"], prr, gamma, beta)
    return sc_out[0, 0] + total_l_rq
